# baseline (speedup 1.0000x reference)
import functools

import jax
import jax.numpy as jnp
from jax import lax
from jax.experimental import pallas as pl
from jax.experimental.pallas import tpu as pltpu
from jax.experimental.pallas import tpu_sc as plsc

N = 10000
E = 160000
D = 128
R = 32
L = 6
BS = 2
K = 16

NC = 2
NS = 16
NW = NC * NS
CH = 128
NCHUNK = 48
TCAP = NCHUNK * CH
NP = 10240
NHALF = NP // 2
RSUB = NHALF // NS
RB = 2000


def _sc_agg_body(h0_hbm, h1_hbm, rel0_hbm, rel1_hbm, src_hbm, dst_hbm, et_hbm,
                 part_hbm, src_vm, dst_vm, et_vm, g0_vm, g1_vm, r0_vm, r1_vm,
                 rel_sh, agg_sh, sg0, sg1, sr0, sr1):
    c = lax.axis_index("c")
    s = lax.axis_index("s")
    slab = c * NS + s

    pltpu.sync_copy(src_hbm.at[slab], src_vm)
    pltpu.sync_copy(dst_hbm.at[slab], dst_vm)
    pltpu.sync_copy(et_hbm.at[slab], et_vm)

    gbuf = (g0_vm, g1_vm)
    rbuf = (r0_vm, r1_vm)
    sg = (sg0, sg1)
    sr = (sr0, sr1)
    row0 = s * RSUB

    def _mul(gb, rb):
        def _mulrow(i, cc):
            for k in range(D // 16):
                sl = pl.ds(k * 16, 16)
                gb[i, sl] = gb[i, sl] * rb[i, sl]
            return cc
        lax.fori_loop(0, CH, _mulrow, 0, unroll=2)

    for b, (h_hbm, rel_hbm) in enumerate(((h0_hbm, rel0_hbm),
                                          (h1_hbm, rel1_hbm))):
        def _zrow(i, carry):
            for k in range(D // 16):
                r0_vm[i, pl.ds(k * 16, 16)] = jnp.zeros((16,), jnp.float32)
            return carry
        lax.fori_loop(0, CH, _zrow, 0, unroll=2)
        for t in range(RSUB // CH):
            pltpu.sync_copy(r0_vm, agg_sh.at[pl.ds(row0 + t * CH, CH)])
        rem = RSUB % CH
        if rem:
            pltpu.sync_copy(r0_vm.at[pl.ds(0, rem)],
                            agg_sh.at[pl.ds(row0 + (RSUB // CH) * CH, rem)])
        @pl.when(s == 0)
        def _stage_rel():
            pltpu.sync_copy(rel_hbm, rel_sh)
        plsc.subcore_barrier()

        pltpu.async_copy(h_hbm.at[src_vm.at[0]], g0_vm, sg0)
        pltpu.async_copy(rel_sh.at[et_vm.at[0]], r0_vm, sr0)

        def _pair(j2, carry):
            j = j2 * 2
            for p in range(2):
                jj = j + p
                gb, rb = gbuf[p], rbuf[p]
                gn, rn = gbuf[1 - p], rbuf[1 - p]
                pltpu.make_async_copy(h_hbm.at[src_vm.at[jj]], gb, sg[p]).wait()
                pltpu.make_async_copy(rel_sh.at[et_vm.at[jj]], rb, sr[p]).wait()
                nxt = jj + 1

                @pl.when(nxt < NCHUNK)
                def _issue_next():
                    pltpu.async_copy(h_hbm.at[src_vm.at[nxt]], gn, sg[1 - p])
                    pltpu.async_copy(rel_sh.at[et_vm.at[nxt]], rn, sr[1 - p])
                _mul(gb, rb)
                pltpu.sync_copy(gb, agg_sh.at[dst_vm.at[jj]], add=True)
            return carry
        lax.fori_loop(0, NCHUNK // 2, _pair, 0)
        plsc.subcore_barrier()

        pltpu.sync_copy(agg_sh.at[pl.ds(row0, RSUB)],
                        part_hbm.at[b, pl.ds(c * NHALF + row0, RSUB)])
        plsc.subcore_barrier()


_sc_agg = functools.partial(
    pl.kernel,
    out_type=jax.ShapeDtypeStruct((BS, NP, D), jnp.float32),
    mesh=plsc.VectorSubcoreMesh(core_axis_name="c", subcore_axis_name="s"),
    scratch_types=[
        pltpu.VMEM((NCHUNK, CH), jnp.int32),
        pltpu.VMEM((NCHUNK, CH), jnp.int32),
        pltpu.VMEM((NCHUNK, CH), jnp.int32),
        pltpu.VMEM((CH, D), jnp.float32),
        pltpu.VMEM((CH, D), jnp.float32),
        pltpu.VMEM((CH, D), jnp.float32),
        pltpu.VMEM((CH, D), jnp.float32),
        pltpu.VMEM_SHARED((R + 1, D), jnp.float32),
        pltpu.VMEM_SHARED((NHALF + 128, D), jnp.float32),
        pltpu.SemaphoreType.DMA,
        pltpu.SemaphoreType.DMA,
        pltpu.SemaphoreType.DMA,
        pltpu.SemaphoreType.DMA,
    ],
)(_sc_agg_body)


def _sc_agg0_body(rel0_hbm, rel1_hbm, q_hbm, dst0_hbm, dst1_hbm, et_hbm,
                  part_hbm, dst_vm, et_vm, q_vm, r0_vm, r1_vm,
                  rel_sh, agg_sh, sr0, sr1):
    c = lax.axis_index("c")
    s = lax.axis_index("s")
    slab = c * NS + s

    pltpu.sync_copy(et_hbm.at[slab], et_vm)
    pltpu.sync_copy(q_hbm, q_vm)

    rbuf = (r0_vm, r1_vm)
    sr = (sr0, sr1)
    row0 = s * RSUB

    for b, (rel_hbm, dstb_hbm) in enumerate(((rel0_hbm, dst0_hbm),
                                             (rel1_hbm, dst1_hbm))):
        pltpu.sync_copy(dstb_hbm.at[slab], dst_vm)

        def _zrow(i, carry):
            for k in range(D // 16):
                r0_vm[i, pl.ds(k * 16, 16)] = jnp.zeros((16,), jnp.float32)
            return carry
        lax.fori_loop(0, CH, _zrow, 0, unroll=2)
        for t in range(RSUB // CH):
            pltpu.sync_copy(r0_vm, agg_sh.at[pl.ds(row0 + t * CH, CH)])
        rem = RSUB % CH
        if rem:
            pltpu.sync_copy(r0_vm.at[pl.ds(0, rem)],
                            agg_sh.at[pl.ds(row0 + (RSUB // CH) * CH, rem)])
        @pl.when(s == 0)
        def _stage_rel():
            pltpu.sync_copy(rel_hbm, rel_sh)
        plsc.subcore_barrier()

        pltpu.async_copy(rel_sh.at[et_vm.at[0]], r0_vm, sr0)

        def _pair(j2, carry):
            j = j2 * 2
            for p in range(2):
                jj = j + p
                rb, rn = rbuf[p], rbuf[1 - p]
                pltpu.make_async_copy(rel_sh.at[et_vm.at[jj]], rb, sr[p]).wait()
                nxt = jj + 1

                @pl.when(nxt < NCHUNK)
                def _issue_next():
                    pltpu.async_copy(rel_sh.at[et_vm.at[nxt]], rn, sr[1 - p])

                def _mulrow(i, cc):
                    for k in range(D // 16):
                        sl = pl.ds(k * 16, 16)
                        rb[i, sl] = rb[i, sl] * q_vm[b, pl.ds(k * 16, 16)]
                    return cc
                lax.fori_loop(0, CH, _mulrow, 0, unroll=2)
                pltpu.sync_copy(rb, agg_sh.at[dst_vm.at[jj]], add=True)
            return carry
        lax.fori_loop(0, NCHUNK // 2, _pair, 0)
        plsc.subcore_barrier()

        pltpu.sync_copy(agg_sh.at[pl.ds(row0, RSUB)],
                        part_hbm.at[b, pl.ds(c * NHALF + row0, RSUB)])
        plsc.subcore_barrier()


_sc_agg0 = functools.partial(
    pl.kernel,
    out_type=jax.ShapeDtypeStruct((BS, NP, D), jnp.float32),
    mesh=plsc.VectorSubcoreMesh(core_axis_name="c", subcore_axis_name="s"),
    scratch_types=[
        pltpu.VMEM((NCHUNK, CH), jnp.int32),
        pltpu.VMEM((NCHUNK, CH), jnp.int32),
        pltpu.VMEM((BS, D), jnp.float32),
        pltpu.VMEM((CH, D), jnp.float32),
        pltpu.VMEM((CH, D), jnp.float32),
        pltpu.VMEM_SHARED((R + 1, D), jnp.float32),
        pltpu.VMEM_SHARED((NHALF + 128, D), jnp.float32),
        pltpu.SemaphoreType.DMA,
        pltpu.SemaphoreType.DMA,
    ],
)(_sc_agg0_body)


def _rel_body(r0_ref, q_ref, wrel_ref, relout_ref, query_ref):
    for b in range(BS):
        rb = r0_ref[b]
        qb = q_ref[pl.ds(rb, 1), :]
        res = jnp.dot(qb, wrel_ref[0], preferred_element_type=jnp.float32)
        for r in range(R):
            relout_ref[0, b, pl.ds(r, 1), :] = res[:, r * D:(r + 1) * D]
        relout_ref[0, b, pl.ds(R, 1), :] = jnp.zeros((1, D), jnp.float32)
        query_ref[pl.ds(b, 1), :] = qb


def _rel_precompute(r0, Q, Wrel):
    return pl.pallas_call(
        _rel_body,
        grid=(L,),
        in_specs=[
            pl.BlockSpec(memory_space=pltpu.SMEM),
            pl.BlockSpec((R, D), lambda l: (0, 0)),
            pl.BlockSpec((1, D, R * D), lambda l: (l, 0, 0)),
        ],
        out_specs=[
            pl.BlockSpec((1, BS, R + 1, D), lambda l: (l, 0, 0, 0)),
            pl.BlockSpec((BS, D), lambda l: (0, 0)),
        ],
        out_shape=[
            jax.ShapeDtypeStruct((L, BS, R + 1, D), jnp.float32),
            jax.ShapeDtypeStruct((BS, D), jnp.float32),
        ],
    )(r0, Q, Wrel)


def _upd_body(h0_ref, part_ref, hid_ref, w_ref, bias_ref, query_ref, out_ref):
    b = pl.program_id(0)
    i = pl.program_id(1)
    agg = part_ref[0]
    rows = lax.broadcasted_iota(jnp.int32, (RB, D), 0) + i * RB
    qb = query_ref[pl.ds(b, 1), :]
    agg = agg + jnp.where(rows == h0_ref[b], qb, 0.0)
    h = hid_ref[0]
    x = jnp.dot(jnp.concatenate([h, agg], axis=1), w_ref[...],
                preferred_element_type=jnp.float32)
    x = jnp.maximum(x + bias_ref[:], 0.0)
    out_ref[0] = x + h


def _update(h0, part, hidden, w, bias, query):
    return pl.pallas_call(
        _upd_body,
        grid=(BS, N // RB),
        in_specs=[
            pl.BlockSpec(memory_space=pltpu.SMEM),
            pl.BlockSpec((1, RB, D), lambda b, i: (b, i, 0)),
            pl.BlockSpec((1, RB, D), lambda b, i: (b, i, 0)),
            pl.BlockSpec((2 * D, D), lambda b, i: (0, 0)),
            pl.BlockSpec((1, D), lambda b, i: (0, 0)),
            pl.BlockSpec((BS, D), lambda b, i: (0, 0)),
        ],
        out_specs=pl.BlockSpec((1, RB, D), lambda b, i: (b, i, 0)),
        out_shape=jax.ShapeDtypeStruct((BS, N, D), jnp.float32),
    )(h0, part, hidden, w, bias, query)


def _gather_body(t_ref, hid_ref, out_ref):
    out_ref[...] = hid_ref[...]


def _gather_rows(t_index, hidden):
    return pl.pallas_call(
        _gather_body,
        grid_spec=pltpu.PrefetchScalarGridSpec(
            num_scalar_prefetch=1,
            grid=(BS, K),
            in_specs=[
                pl.BlockSpec((1, 1, 1, D), lambda b, k, t: (b, t[b, k], 0, 0)),
            ],
            out_specs=pl.BlockSpec((1, 1, 1, D), lambda b, k, t: (b, k, 0, 0)),
        ),
        out_shape=jax.ShapeDtypeStruct((BS, K, 1, D), jnp.float32),
    )(t_index, hidden.reshape(BS, N, 1, D))


def _mlp_body(g_ref, query_ref, w1_ref, b1_ref, w2_ref, b2_ref, out_ref):
    g = g_ref[...]
    rows = lax.broadcasted_iota(jnp.int32, (BS * K, D), 0)
    q0 = query_ref[pl.ds(0, 1), :]
    q1 = query_ref[pl.ds(1, 1), :]
    q = jnp.where(rows < K, q0, q1)
    feat = jnp.concatenate([g, q], axis=1)
    h1 = jnp.dot(feat, w1_ref[...], preferred_element_type=jnp.float32)
    h1 = jnp.maximum(h1 + b1_ref[:], 0.0)
    sc = jnp.dot(h1, w2_ref[...], preferred_element_type=jnp.float32)
    out_ref[...] = sc[:, 0:1] + b2_ref[0, 0]


def _score_mlp(g, query, W1, b1row, w2row, b2s):
    return pl.pallas_call(
        _mlp_body,
        in_specs=[
            pl.BlockSpec((BS * K, D), lambda: (0, 0)),
            pl.BlockSpec((BS, D), lambda: (0, 0)),
            pl.BlockSpec((2 * D, 2 * D), lambda: (0, 0)),
            pl.BlockSpec((1, 2 * D), lambda: (0, 0)),
            pl.BlockSpec((2 * D, D), lambda: (0, 0)),
            pl.BlockSpec(memory_space=pltpu.SMEM),
        ],
        out_specs=pl.BlockSpec((BS * K, 1), lambda: (0, 0)),
        out_shape=jax.ShapeDtypeStruct((BS * K, 1), jnp.float32),
    )(g, query, W1, b1row, w2row, b2s)


def kernel(edge_index, edge_type, h_index, t_index, r_index, Q, Wrel, Wupd,
           bupd, W1, b1, W2, b2):
    src = edge_index[0].astype(jnp.int32)
    dst = edge_index[1].astype(jnp.int32)
    et = edge_type.astype(jnp.int32)

    grp = dst // RSUB
    oh = (grp[:, None] == jnp.arange(NW, dtype=jnp.int32)[None, :]).astype(jnp.int32)
    rank = jnp.take_along_axis(jnp.cumsum(oh, axis=0), grp[:, None], axis=1)[:, 0] - 1
    pos = grp * TCAP + rank
    dstl = dst - (grp // NS) * NHALF
    slots = jnp.arange(NW * TCAP, dtype=jnp.int32)
    zero_bank = jnp.zeros((NW * TCAP,), jnp.int32)
    src_b = zero_bank.at[pos].add(src + 1, unique_indices=True)
    dst_b = zero_bank.at[pos].add(dstl + 1, unique_indices=True)
    et_b = zero_bank.at[pos].add(et + 1, unique_indices=True)
    empty = src_b == 0
    slot_g = slots // TCAP
    fill_dst = (slot_g % NS) * RSUB + slots % RSUB
    src_t = jnp.where(empty, (slots * 37) % N, src_b - 1).reshape(NW, NCHUNK, CH)
    dst_t = jnp.where(empty, fill_dst, dst_b - 1).reshape(NW, NCHUNK, CH)
    et_t = jnp.where(empty, R, et_b - 1).reshape(NW, NCHUNK, CH)

    r0 = r_index[:, 0].astype(jnp.int32)
    h0 = h_index[:, 0].astype(jnp.int32)

    trash = NHALF + (grp % NS) * 8 + (jnp.arange(E, dtype=jnp.int32) % 8)
    fill_trash = NHALF + (slot_g % NS) * 8 + slots % 8
    dst_eff = []
    for b in range(BS):
        de = jnp.where(src == h0[b], dstl, trash)
        de_b = zero_bank.at[pos].add(de + 1, unique_indices=True)
        dst_eff.append(jnp.where(empty, fill_trash, de_b - 1)
                       .reshape(NW, NCHUNK, CH))

    rel_all, query = _rel_precompute(r0, Q, Wrel)

    hidden = jnp.zeros((BS, N, D), jnp.float32).at[jnp.arange(BS), h0].add(query)

    for l in range(L):
        if l == 0:
            part = _sc_agg0(rel_all[0, 0], rel_all[0, 1], query,
                            dst_eff[0], dst_eff[1], et_t)
        else:
            part = _sc_agg(hidden[0], hidden[1], rel_all[l, 0], rel_all[l, 1],
                           src_t, dst_t, et_t)
        hidden = _update(h0, part, hidden, Wupd[l],
                         bupd[l].reshape(1, D), query)

    g = _gather_rows(t_index.astype(jnp.int32), hidden).reshape(BS * K, D)
    w2mat = jnp.zeros((2 * D, D), jnp.float32).at[:, 0].set(W2[:, 0])
    score = _score_mlp(g, query, W1, b1.reshape(1, 2 * D),
                       w2mat, b2.reshape(1, 1))
    return score.reshape(BS, K)


# speedup vs baseline: 1.0041x; 1.0041x over previous
import functools

import jax
import jax.numpy as jnp
from jax import lax
from jax.experimental import pallas as pl
from jax.experimental.pallas import tpu as pltpu
from jax.experimental.pallas import tpu_sc as plsc

N = 10000
E = 160000
D = 128
R = 32
L = 6
BS = 2
K = 16

NC = 2
NS = 16
NW = NC * NS
CH = 128
NCHUNK = 48
TCAP = NCHUNK * CH
NP = 10240
NHALF = NP // 2
RSUB = NHALF // NS
RB = 2000


def _sc_agg_body(h0_hbm, h1_hbm, rel0_hbm, rel1_hbm, src_hbm, dst_hbm, et_hbm,
                 part_hbm, src_vm, dst_vm, et_vm, g0_vm, g1_vm, r0_vm, r1_vm,
                 rel_sh, agg_sh, sg0, sg1, sr0, sr1):
    c = lax.axis_index("c")
    s = lax.axis_index("s")
    slab = c * NS + s

    pltpu.sync_copy(src_hbm.at[slab], src_vm)
    pltpu.sync_copy(dst_hbm.at[slab], dst_vm)
    pltpu.sync_copy(et_hbm.at[slab], et_vm)

    gbuf = (g0_vm, g1_vm)
    rbuf = (r0_vm, r1_vm)
    sg = (sg0, sg1)
    sr = (sr0, sr1)
    row0 = s * RSUB

    def _mul(gb, rb):
        def _mulrow(i, cc):
            for k in range(D // 16):
                sl = pl.ds(k * 16, 16)
                gb[i, sl] = gb[i, sl] * rb[i, sl]
            return cc
        lax.fori_loop(0, CH, _mulrow, 0, unroll=2)

    for b, (h_hbm, rel_hbm) in enumerate(((h0_hbm, rel0_hbm),
                                          (h1_hbm, rel1_hbm))):
        def _zrow(i, carry):
            for k in range(D // 16):
                r0_vm[i, pl.ds(k * 16, 16)] = jnp.zeros((16,), jnp.float32)
            return carry
        lax.fori_loop(0, CH, _zrow, 0, unroll=2)
        for t in range(RSUB // CH):
            pltpu.sync_copy(r0_vm, agg_sh.at[pl.ds(row0 + t * CH, CH)])
        rem = RSUB % CH
        if rem:
            pltpu.sync_copy(r0_vm.at[pl.ds(0, rem)],
                            agg_sh.at[pl.ds(row0 + (RSUB // CH) * CH, rem)])
        @pl.when(s == 0)
        def _stage_rel():
            pltpu.sync_copy(rel_hbm, rel_sh)
        plsc.subcore_barrier()

        pltpu.async_copy(h_hbm.at[src_vm.at[0]], g0_vm, sg0)
        pltpu.async_copy(rel_sh.at[et_vm.at[0]], r0_vm, sr0)

        def _pair(j2, carry):
            j = j2 * 2
            for p in range(2):
                jj = j + p
                gb, rb = gbuf[p], rbuf[p]
                gn, rn = gbuf[1 - p], rbuf[1 - p]
                pltpu.make_async_copy(h_hbm.at[src_vm.at[jj]], gb, sg[p]).wait()
                pltpu.make_async_copy(rel_sh.at[et_vm.at[jj]], rb, sr[p]).wait()
                nxt = jj + 1

                @pl.when(nxt < NCHUNK)
                def _issue_next():
                    pltpu.async_copy(h_hbm.at[src_vm.at[nxt]], gn, sg[1 - p])
                    pltpu.async_copy(rel_sh.at[et_vm.at[nxt]], rn, sr[1 - p])
                _mul(gb, rb)
                pltpu.sync_copy(gb, agg_sh.at[dst_vm.at[jj]], add=True)
            return carry
        lax.fori_loop(0, NCHUNK // 2, _pair, 0)
        plsc.subcore_barrier()

        pltpu.sync_copy(agg_sh.at[pl.ds(row0, RSUB)],
                        part_hbm.at[b, pl.ds(c * NHALF + row0, RSUB)])
        plsc.subcore_barrier()


_sc_agg = functools.partial(
    pl.kernel,
    out_type=jax.ShapeDtypeStruct((BS, NP, D), jnp.float32),
    mesh=plsc.VectorSubcoreMesh(core_axis_name="c", subcore_axis_name="s"),
    scratch_types=[
        pltpu.VMEM((NCHUNK, CH), jnp.int32),
        pltpu.VMEM((NCHUNK, CH), jnp.int32),
        pltpu.VMEM((NCHUNK, CH), jnp.int32),
        pltpu.VMEM((CH, D), jnp.float32),
        pltpu.VMEM((CH, D), jnp.float32),
        pltpu.VMEM((CH, D), jnp.float32),
        pltpu.VMEM((CH, D), jnp.float32),
        pltpu.VMEM_SHARED((R + 1, D), jnp.float32),
        pltpu.VMEM_SHARED((NHALF, D), jnp.float32),
        pltpu.SemaphoreType.DMA,
        pltpu.SemaphoreType.DMA,
        pltpu.SemaphoreType.DMA,
        pltpu.SemaphoreType.DMA,
    ],
)(_sc_agg_body)


def _rel_body(r0_ref, q_ref, wrel_ref, relout_ref, query_ref):
    for b in range(BS):
        rb = r0_ref[b]
        qb = q_ref[pl.ds(rb, 1), :]
        res = jnp.dot(qb, wrel_ref[0], preferred_element_type=jnp.float32)
        for r in range(R):
            relout_ref[0, b, pl.ds(r, 1), :] = res[:, r * D:(r + 1) * D]
        relout_ref[0, b, pl.ds(R, 1), :] = jnp.zeros((1, D), jnp.float32)
        query_ref[pl.ds(b, 1), :] = qb


def _rel_precompute(r0, Q, Wrel):
    return pl.pallas_call(
        _rel_body,
        grid=(L,),
        in_specs=[
            pl.BlockSpec(memory_space=pltpu.SMEM),
            pl.BlockSpec((R, D), lambda l: (0, 0)),
            pl.BlockSpec((1, D, R * D), lambda l: (l, 0, 0)),
        ],
        out_specs=[
            pl.BlockSpec((1, BS, R + 1, D), lambda l: (l, 0, 0, 0)),
            pl.BlockSpec((BS, D), lambda l: (0, 0)),
        ],
        out_shape=[
            jax.ShapeDtypeStruct((L, BS, R + 1, D), jnp.float32),
            jax.ShapeDtypeStruct((BS, D), jnp.float32),
        ],
    )(r0, Q, Wrel)


def _upd_body(h0_ref, part_ref, hid_ref, w_ref, bias_ref, query_ref, out_ref):
    b = pl.program_id(0)
    i = pl.program_id(1)
    agg = part_ref[0]
    rows = lax.broadcasted_iota(jnp.int32, (RB, D), 0) + i * RB
    qb = query_ref[pl.ds(b, 1), :]
    agg = agg + jnp.where(rows == h0_ref[b], qb, 0.0)
    h = hid_ref[0]
    x = jnp.dot(jnp.concatenate([h, agg], axis=1), w_ref[...],
                preferred_element_type=jnp.float32)
    x = jnp.maximum(x + bias_ref[:], 0.0)
    out_ref[0] = x + h


def _update(h0, part, hidden, w, bias, query):
    return pl.pallas_call(
        _upd_body,
        grid=(BS, N // RB),
        in_specs=[
            pl.BlockSpec(memory_space=pltpu.SMEM),
            pl.BlockSpec((1, RB, D), lambda b, i: (b, i, 0)),
            pl.BlockSpec((1, RB, D), lambda b, i: (b, i, 0)),
            pl.BlockSpec((2 * D, D), lambda b, i: (0, 0)),
            pl.BlockSpec((1, D), lambda b, i: (0, 0)),
            pl.BlockSpec((BS, D), lambda b, i: (0, 0)),
        ],
        out_specs=pl.BlockSpec((1, RB, D), lambda b, i: (b, i, 0)),
        out_shape=jax.ShapeDtypeStruct((BS, N, D), jnp.float32),
    )(h0, part, hidden, w, bias, query)


def _gather_body(t_ref, hid_ref, out_ref):
    out_ref[...] = hid_ref[...]


def _gather_rows(t_index, hidden):
    return pl.pallas_call(
        _gather_body,
        grid_spec=pltpu.PrefetchScalarGridSpec(
            num_scalar_prefetch=1,
            grid=(BS, K),
            in_specs=[
                pl.BlockSpec((1, 1, 1, D), lambda b, k, t: (b, t[b, k], 0, 0)),
            ],
            out_specs=pl.BlockSpec((1, 1, 1, D), lambda b, k, t: (b, k, 0, 0)),
        ),
        out_shape=jax.ShapeDtypeStruct((BS, K, 1, D), jnp.float32),
    )(t_index, hidden.reshape(BS, N, 1, D))


def _mlp_body(g_ref, query_ref, w1_ref, b1_ref, w2_ref, b2_ref, out_ref):
    g = g_ref[...]
    rows = lax.broadcasted_iota(jnp.int32, (BS * K, D), 0)
    q0 = query_ref[pl.ds(0, 1), :]
    q1 = query_ref[pl.ds(1, 1), :]
    q = jnp.where(rows < K, q0, q1)
    feat = jnp.concatenate([g, q], axis=1)
    h1 = jnp.dot(feat, w1_ref[...], preferred_element_type=jnp.float32)
    h1 = jnp.maximum(h1 + b1_ref[:], 0.0)
    sc = jnp.dot(h1, w2_ref[...], preferred_element_type=jnp.float32)
    out_ref[...] = sc[:, 0:1] + b2_ref[0, 0]


def _score_mlp(g, query, W1, b1row, w2row, b2s):
    return pl.pallas_call(
        _mlp_body,
        in_specs=[
            pl.BlockSpec((BS * K, D), lambda: (0, 0)),
            pl.BlockSpec((BS, D), lambda: (0, 0)),
            pl.BlockSpec((2 * D, 2 * D), lambda: (0, 0)),
            pl.BlockSpec((1, 2 * D), lambda: (0, 0)),
            pl.BlockSpec((2 * D, D), lambda: (0, 0)),
            pl.BlockSpec(memory_space=pltpu.SMEM),
        ],
        out_specs=pl.BlockSpec((BS * K, 1), lambda: (0, 0)),
        out_shape=jax.ShapeDtypeStruct((BS * K, 1), jnp.float32),
    )(g, query, W1, b1row, w2row, b2s)


def kernel(edge_index, edge_type, h_index, t_index, r_index, Q, Wrel, Wupd,
           bupd, W1, b1, W2, b2):
    src = edge_index[0].astype(jnp.int32)
    dst = edge_index[1].astype(jnp.int32)
    et = edge_type.astype(jnp.int32)

    grp = dst // RSUB
    oh = (grp[:, None] == jnp.arange(NW, dtype=jnp.int32)[None, :]).astype(jnp.int32)
    rank = jnp.take_along_axis(jnp.cumsum(oh, axis=0), grp[:, None], axis=1)[:, 0] - 1
    pos = grp * TCAP + rank
    dstl = dst - (grp // NS) * NHALF
    slots = jnp.arange(NW * TCAP, dtype=jnp.int32)
    zero_bank = jnp.zeros((NW * TCAP,), jnp.int32)
    src_b = zero_bank.at[pos].add(src + 1, unique_indices=True)
    dst_b = zero_bank.at[pos].add(dstl + 1, unique_indices=True)
    et_b = zero_bank.at[pos].add(et + 1, unique_indices=True)
    empty = src_b == 0
    slot_g = slots // TCAP
    fill_dst = (slot_g % NS) * RSUB + slots % RSUB
    src_t = jnp.where(empty, (slots * 37) % N, src_b - 1).reshape(NW, NCHUNK, CH)
    dst_t = jnp.where(empty, fill_dst, dst_b - 1).reshape(NW, NCHUNK, CH)
    et_t = jnp.where(empty, R, et_b - 1).reshape(NW, NCHUNK, CH)

    r0 = r_index[:, 0].astype(jnp.int32)
    h0 = h_index[:, 0].astype(jnp.int32)

    rel_all, query = _rel_precompute(r0, Q, Wrel)

    hidden = jnp.zeros((BS, N, D), jnp.float32).at[jnp.arange(BS), h0].add(query)

    for l in range(L):
        part = _sc_agg(hidden[0], hidden[1], rel_all[l, 0], rel_all[l, 1],
                       src_t, dst_t, et_t)
        hidden = _update(h0, part, hidden, Wupd[l],
                         bupd[l].reshape(1, D), query)

    g = _gather_rows(t_index.astype(jnp.int32), hidden).reshape(BS * K, D)
    w2mat = jnp.zeros((2 * D, D), jnp.float32).at[:, 0].set(W2[:, 0])
    score = _score_mlp(g, query, W1, b1.reshape(1, 2 * D),
                       w2mat, b2.reshape(1, 1))
    return score.reshape(BS, K)
